# Initial kernel scaffold; baseline (speedup 1.0000x reference)
import jax
import jax.numpy as jnp
from jax.experimental import pallas as pl


def kernel(x, edge_index, edge_attr, batch, nn1_W1, nn1_b1, nn1_W2, nn1_b2, root1, bias1, nn2_W1, nn2_b1, nn2_W2, nn2_b2, root2, bias2, nn3_W1, nn3_b1, nn3_W2, nn3_b2, root3, bias3, fc1_W, fc1_b, fc2_W, fc2_b, fc3_W, fc3_b):
    raise NotImplementedError("write your pallas kernel here")


# speedup vs baseline: 230.3279x; 230.3279x over previous
import functools

import jax
import jax.numpy as jnp
from jax import lax
from jax.experimental import pallas as pl
from jax.experimental.pallas import tpu as pltpu
from jax.experimental.pallas import tpu_sc as plsc

_N = 10240
_E = 20480
_FEAT = 32
_RUNS = 2
_P = 2.0 / (1.0 + _RUNS)
_NGRAPH = 512
_NC = 2
_NS = 16
_NW = _NC * _NS
_CHUNK = 128
_EDGES_ALL = _RUNS * _E
_EPW = _EDGES_ALL // _NW
_J = _EPW // _CHUNK
_T = _RUNS * _N
_STRIPE = _T // _NS


def _i32map(*vals):
    return tuple(jnp.int32(v) for v in vals)


@functools.cache
def _gather_fn(cin):
    mesh = plsc.VectorSubcoreMesh(core_axis_name="c", subcore_axis_name="s")

    @functools.partial(
        pl.kernel,
        out_type=jax.ShapeDtypeStruct((_NW, _J, _CHUNK, cin), jnp.float32),
        mesh=mesh,
        compiler_params=pltpu.CompilerParams(use_tc_tiling_on_sc=False),
        scratch_types=[
            pltpu.VMEM((_J, _CHUNK), jnp.int32),
            pltpu.VMEM((_J, _CHUNK, cin), jnp.float32),
            pltpu.SemaphoreType.DMA,
        ],
    )
    def gather(table_hbm, idx_hbm, out_hbm, idx_v, rows_v, sem):
        wid = lax.axis_index("s") * _NC + lax.axis_index("c")
        pltpu.sync_copy(idx_hbm.at[wid], idx_v)
        descs = [
            pltpu.async_copy(
                table_hbm.at[idx_v.at[jnp.int32(j)]], rows_v.at[jnp.int32(j)], sem)
            for j in range(_J)
        ]
        for d in descs:
            d.wait()
        pltpu.sync_copy(rows_v, out_hbm.at[wid])

    return gather


_HALF = _T // _NC
_J2 = _EDGES_ALL // _NS // _CHUNK
_STRIPE2 = _HALF // _NS


@functools.cache
def _scatter_fn(cout):
    mesh = plsc.VectorSubcoreMesh(core_axis_name="c", subcore_axis_name="s")

    @functools.partial(
        pl.kernel,
        out_type=jax.ShapeDtypeStruct((_T, cout), jnp.float32),
        mesh=mesh,
        compiler_params=pltpu.CompilerParams(use_tc_tiling_on_sc=False),
        scratch_types=[
            pltpu.VMEM((_J2, _CHUNK), jnp.int32),
            pltpu.VMEM((_CHUNK, cout), jnp.float32),
            pltpu.VMEM_SHARED((_HALF + 8, cout), jnp.float32),
            pltpu.SemaphoreType.DMA,
        ],
    )
    def scatter(msg_hbm, idx_hbm, zeros_hbm, out_hbm, idx_v, msg_v, acc_sh, sem):
        c = lax.axis_index("c")
        s = lax.axis_index("s")
        base = c * _HALF
        pltpu.sync_copy(zeros_hbm, acc_sh.at[pl.ds(s * _STRIPE2, _STRIPE2)])
        pltpu.sync_copy(idx_hbm.at[s], idx_v)
        for j in range(_J2):
            for i in range(_CHUNK // 16):
                v = idx_v[jnp.int32(j), pl.ds(i * 16, 16)] - base
                ok = (v >= 0) & (v < _HALF)
                idx_v[jnp.int32(j), pl.ds(i * 16, 16)] = jnp.where(
                    ok, v, jnp.int32(_HALF))
        plsc.subcore_barrier()
        for j in range(_J2):
            pltpu.sync_copy(msg_hbm.at[s * _J2 + j], msg_v)
            pltpu.async_copy(
                msg_v, acc_sh.at[idx_v.at[jnp.int32(j)]], sem, add=True
            ).wait()
        plsc.subcore_barrier()
        pltpu.sync_copy(
            acc_sh.at[pl.ds(s * _STRIPE2, _STRIPE2)],
            out_hbm.at[pl.ds(base + s * _STRIPE2, _STRIPE2)],
        )

    return scatter


def _msg_body(ea_ref, w1_ref, b1_ref, w2_ref, b2_ref, xj0_ref, xj1_ref, out_ref,
              *, cin, cout, be):
    gi = 128 // cout
    nchunk = cin // gi
    h1 = jnp.maximum(
        jnp.dot(ea_ref[...], w1_ref[...], preferred_element_type=jnp.float32,
                    precision=lax.Precision.HIGHEST)
        + b1_ref[...],
        0.0,
    )
    xj0 = xj0_ref[0]
    xj1 = xj1_ref[0]
    lane = lax.broadcasted_iota(jnp.int32, (be, 128), 1)
    acc0 = jnp.zeros((be, 128), jnp.float32)
    acc1 = jnp.zeros((be, 128), jnp.float32)
    for g in range(nchunk):
        wg = (
            jnp.dot(h1, w2_ref[:, g * 128:(g + 1) * 128],
                    preferred_element_type=jnp.float32,
                    precision=lax.Precision.HIGHEST)
            + b2_ref[:, g * 128:(g + 1) * 128]
        )

        def sel(xj):
            i0 = g * gi
            v = xj[:, i0 + gi - 1:i0 + gi]
            for t in range(gi - 2, -1, -1):
                v = jnp.where(lane < (t + 1) * cout, xj[:, i0 + t:i0 + t + 1], v)
            return v

        acc0 += sel(xj0) * wg
        acc1 += sel(xj1) * wg
    w = 128
    while w > cout:
        w //= 2
        acc0 = acc0[:, :w] + acc0[:, w:2 * w]
        acc1 = acc1[:, :w] + acc1[:, w:2 * w]
    out_ref[0] = acc0
    out_ref[1] = acc1


def _msg_call(ea, w1, b1, w2, b2, xj, cin, cout, be=256):
    nblk = _E // be
    body = functools.partial(_msg_body, cin=cin, cout=cout, be=be)
    return pl.pallas_call(
        body,
        grid=(nblk,),
        in_specs=[
            pl.BlockSpec((be, 5), lambda k: _i32map(k, 0)),
            pl.BlockSpec((5, 128), lambda k: _i32map(0, 0)),
            pl.BlockSpec((1, 128), lambda k: _i32map(0, 0)),
            pl.BlockSpec((128, cin * cout), lambda k: _i32map(0, 0)),
            pl.BlockSpec((1, cin * cout), lambda k: _i32map(0, 0)),
            pl.BlockSpec((1, be, cin), lambda k: _i32map(0, k, 0)),
            pl.BlockSpec((1, be, cin), lambda k: _i32map(1, k, 0)),
        ],
        out_specs=pl.BlockSpec((2, be, cout), lambda k: _i32map(0, k, 0)),
        out_shape=jax.ShapeDtypeStruct((2, _E, cout), jnp.float32),
    )(ea, w1, b1, w2, b2, xj, xj)


def _epi_call(part, hprev, root, bias, cin, cout, bn=2048):
    def body(p_ref, h_ref, r_ref, b_ref, o_ref):
        s = (
            p_ref[...]
            + jnp.dot(h_ref[...], r_ref[...], preferred_element_type=jnp.float32,
                    precision=lax.Precision.HIGHEST)
            + b_ref[...]
        )
        o_ref[...] = jnp.where(s > 0, s, jnp.exp(s) - 1.0)

    return pl.pallas_call(
        body,
        grid=(_T // bn,),
        in_specs=[
            pl.BlockSpec((bn, cout), lambda k: _i32map(k, 0)),
            pl.BlockSpec((bn, cin), lambda k: _i32map(k, 0)),
            pl.BlockSpec((cin, cout), lambda k: _i32map(0, 0)),
            pl.BlockSpec((1, cout), lambda k: _i32map(0, 0)),
        ],
        out_specs=pl.BlockSpec((bn, cout), lambda k: _i32map(k, 0)),
        out_shape=jax.ShapeDtypeStruct((_T, cout), jnp.float32),
    )(part, hprev, root, bias)


def _readout_call(h3, batch3, fc1_W, fc1_b, fc2_W, fc2_b, fc3_W, fc3_b, nb=1024):
    nblk = _N // nb

    def body(h0_ref, h1_ref, b_ref, w1_ref, b1_ref, w2_ref, b2_ref, w3_ref,
             b3_ref, o_ref, acc_ref, cnt_ref):
        k = pl.program_id(0)

        @pl.when(k == 0)
        def _():
            acc_ref[...] = jnp.zeros_like(acc_ref)
            cnt_ref[...] = jnp.zeros_like(cnt_ref)

        hm = (h0_ref[0] + h1_ref[0]) * 0.5
        brow = b_ref[0]
        gids = lax.broadcasted_iota(jnp.int32, (_NGRAPH, nb), 0)
        oh = (gids == brow).astype(jnp.float32)
        acc_ref[...] += jnp.dot(oh, hm, preferred_element_type=jnp.float32,
                    precision=lax.Precision.HIGHEST)
        cnt_ref[...] += jnp.dot(oh, jnp.ones((nb, 64), jnp.float32),
                                preferred_element_type=jnp.float32,
                    precision=lax.Precision.HIGHEST)

        @pl.when(k == nblk - 1)
        def _():
            g = acc_ref[...] / jnp.maximum(cnt_ref[...], 1.0)
            g1 = jnp.dot(g, w1_ref[...], preferred_element_type=jnp.float32,
                    precision=lax.Precision.HIGHEST) + b1_ref[...]
            g1 = jnp.where(g1 > 0, g1, jnp.exp(g1) - 1.0)
            g2 = jnp.dot(g1, w2_ref[...], preferred_element_type=jnp.float32,
                    precision=lax.Precision.HIGHEST) + b2_ref[...]
            g2 = jnp.where(g2 > 0, g2, jnp.exp(g2) - 1.0)
            o_ref[...] = jnp.dot(g2, w3_ref[...], preferred_element_type=jnp.float32,
                    precision=lax.Precision.HIGHEST) + b3_ref[...]

    return pl.pallas_call(
        body,
        grid=(nblk,),
        in_specs=[
            pl.BlockSpec((1, nb, 64), lambda k: _i32map(0, k, 0)),
            pl.BlockSpec((1, nb, 64), lambda k: _i32map(1, k, 0)),
            pl.BlockSpec((1, 1, nb), lambda k: _i32map(k, 0, 0)),
            pl.BlockSpec((64, 32), lambda k: _i32map(0, 0)),
            pl.BlockSpec((1, 32), lambda k: _i32map(0, 0)),
            pl.BlockSpec((32, 16), lambda k: _i32map(0, 0)),
            pl.BlockSpec((1, 16), lambda k: _i32map(0, 0)),
            pl.BlockSpec((16, 1), lambda k: _i32map(0, 0)),
            pl.BlockSpec((1, 1), lambda k: _i32map(0, 0)),
        ],
        out_specs=pl.BlockSpec((_NGRAPH, 1), lambda k: _i32map(0, 0)),
        out_shape=jax.ShapeDtypeStruct((_NGRAPH, 1), jnp.float32),
        scratch_shapes=[
            pltpu.VMEM((_NGRAPH, 64), jnp.float32),
            pltpu.VMEM((_NGRAPH, 64), jnp.float32),
        ],
    )(h3, h3, batch3, fc1_W, fc1_b, fc2_W, fc2_b, fc3_W, fc3_b)


def kernel(x, edge_index, edge_attr, batch, nn1_W1, nn1_b1, nn1_W2, nn1_b2,
           root1, bias1, nn2_W1, nn2_b1, nn2_W2, nn2_b2, root2, bias2,
           nn3_W1, nn3_b1, nn3_W2, nn3_b2, root3, bias3, fc1_W, fc1_b,
           fc2_W, fc2_b, fc3_W, fc3_b):
    f32 = jnp.float32
    ei = edge_index.astype(jnp.int32)
    off = (jnp.max(edge_index) + 1).astype(jnp.int32)
    src = jnp.concatenate([ei[0], ei[0] + off]).reshape(_NW, _J, _CHUNK)
    dst = jnp.concatenate([ei[1], ei[1] + off]).reshape(_NS, _J2, _CHUNK)

    drop = jax.random.bernoulli(jax.random.key(42), _P, (_RUNS, _N))
    xr = jnp.where(
        drop[..., None], 0.0, jnp.broadcast_to(x[None], (_RUNS, _N, _FEAT))
    ).reshape(_T, _FEAT).astype(f32)

    ea = edge_attr.astype(f32)
    zeros32 = jnp.zeros((_STRIPE2, 32), f32)
    zeros64 = jnp.zeros((_STRIPE2, 64), f32)

    def layer(h, W1, b1, W2, b2, root, bias, cin, cout, zeros):
        xjr = _gather_fn(cin)(h, src)
        xj = xjr.reshape(_RUNS, _E, cin)
        msg = _msg_call(ea, W1.astype(f32), b1.reshape(1, 128).astype(f32),
                        W2.astype(f32), b2.reshape(1, -1).astype(f32),
                        xj, cin, cout)
        part = _scatter_fn(cout)(msg.reshape(_NS * _J2, _CHUNK, cout), dst, zeros)
        return _epi_call(part, h, root.astype(f32),
                         bias.reshape(1, cout).astype(f32), cin, cout)

    h1 = layer(xr, nn1_W1, nn1_b1, nn1_W2, nn1_b2, root1, bias1, _FEAT, 32, zeros32)
    h2 = layer(h1, nn2_W1, nn2_b1, nn2_W2, nn2_b2, root2, bias2, 32, 64, zeros64)
    h3 = layer(h2, nn3_W1, nn3_b1, nn3_W2, nn3_b2, root3, bias3, 64, 64, zeros64)

    g = _readout_call(
        h3.reshape(_RUNS, _N, 64),
        batch.astype(jnp.int32).reshape(_N // 1024, 1, 1024),
        fc1_W.astype(f32), fc1_b.reshape(1, 32).astype(f32),
        fc2_W.astype(f32), fc2_b.reshape(1, 16).astype(f32),
        fc3_W.astype(f32), fc3_b.reshape(1, 1).astype(f32),
    )
    return g.reshape(-1).astype(nn1_W1.dtype)


# speedup vs baseline: 232.7580x; 1.0106x over previous
import functools

import jax
import jax.numpy as jnp
from jax import lax
from jax.experimental import pallas as pl
from jax.experimental.pallas import tpu as pltpu
from jax.experimental.pallas import tpu_sc as plsc

_N = 10240
_E = 20480
_FEAT = 32
_RUNS = 2
_P = 2.0 / (1.0 + _RUNS)
_NGRAPH = 512
_NC = 2
_NS = 16
_NW = _NC * _NS
_CHUNK = 128
_EDGES_ALL = _RUNS * _E
_EPW = _EDGES_ALL // _NW
_J = _EPW // _CHUNK
_T = _RUNS * _N
_STRIPE = _T // _NS


def _i32map(*vals):
    return tuple(jnp.int32(v) for v in vals)


@functools.cache
def _gather_fn(cin):
    mesh = plsc.VectorSubcoreMesh(core_axis_name="c", subcore_axis_name="s")

    @functools.partial(
        pl.kernel,
        out_type=jax.ShapeDtypeStruct((_NW, _J, _CHUNK, cin), jnp.float32),
        mesh=mesh,
        compiler_params=pltpu.CompilerParams(use_tc_tiling_on_sc=False),
        scratch_types=[
            pltpu.VMEM((_J, _CHUNK), jnp.int32),
            pltpu.VMEM((_J, _CHUNK, cin), jnp.float32),
            pltpu.SemaphoreType.DMA,
        ],
    )
    def gather(table_hbm, idx_hbm, out_hbm, idx_v, rows_v, sem):
        wid = lax.axis_index("s") * _NC + lax.axis_index("c")
        pltpu.sync_copy(idx_hbm.at[wid], idx_v)
        descs = [
            pltpu.async_copy(
                table_hbm.at[idx_v.at[jnp.int32(j)]], rows_v.at[jnp.int32(j)], sem)
            for j in range(_J)
        ]
        for d in descs:
            d.wait()
        pltpu.sync_copy(rows_v, out_hbm.at[wid])

    return gather


_HALF = _T // _NC
_J2 = _EDGES_ALL // _NS // _CHUNK
_STRIPE2 = _HALF // _NS


@functools.cache
def _scatter_fn(cout):
    mesh = plsc.VectorSubcoreMesh(core_axis_name="c", subcore_axis_name="s")

    @functools.partial(
        pl.kernel,
        out_type=jax.ShapeDtypeStruct((_T, cout), jnp.float32),
        mesh=mesh,
        compiler_params=pltpu.CompilerParams(use_tc_tiling_on_sc=False),
        scratch_types=[
            pltpu.VMEM((_J2, _CHUNK), jnp.int32),
            pltpu.VMEM((_CHUNK, cout), jnp.float32),
            pltpu.VMEM_SHARED((_HALF + 8, cout), jnp.float32),
            pltpu.SemaphoreType.DMA,
        ],
    )
    def scatter(msg_hbm, idx_hbm, zeros_hbm, out_hbm, idx_v, msg_v, acc_sh, sem):
        c = lax.axis_index("c")
        s = lax.axis_index("s")
        base = c * _HALF
        pltpu.sync_copy(zeros_hbm, acc_sh.at[pl.ds(s * _STRIPE2, _STRIPE2)])
        pltpu.sync_copy(idx_hbm.at[s], idx_v)
        for j in range(_J2):
            for i in range(_CHUNK // 16):
                v = idx_v[jnp.int32(j), pl.ds(i * 16, 16)] - base
                ok = (v >= 0) & (v < _HALF)
                idx_v[jnp.int32(j), pl.ds(i * 16, 16)] = jnp.where(
                    ok, v, jnp.int32(_HALF))
        plsc.subcore_barrier()
        for j in range(_J2):
            pltpu.sync_copy(msg_hbm.at[s * _J2 + j], msg_v)
            pltpu.async_copy(
                msg_v, acc_sh.at[idx_v.at[jnp.int32(j)]], sem, add=True
            ).wait()
        plsc.subcore_barrier()
        pltpu.sync_copy(
            acc_sh.at[pl.ds(s * _STRIPE2, _STRIPE2)],
            out_hbm.at[pl.ds(base + s * _STRIPE2, _STRIPE2)],
        )

    return scatter


def _msg_body(ea_ref, w1_ref, b1_ref, w2_ref, b2_ref, xj0_ref, xj1_ref, out_ref,
              *, cin, cout, be):
    gi = 128 // cout
    nchunk = cin // gi
    h1 = jnp.maximum(
        jnp.dot(ea_ref[...], w1_ref[...], preferred_element_type=jnp.float32,
                    precision=lax.Precision.HIGHEST)
        + b1_ref[...],
        0.0,
    )
    h1h = h1.astype(jnp.bfloat16)
    h1l = (h1 - h1h.astype(jnp.float32)).astype(jnp.bfloat16)
    hcat = jnp.concatenate([h1h, h1l, h1h], axis=1)
    xj0 = xj0_ref[0]
    xj1 = xj1_ref[0]
    lane = lax.broadcasted_iota(jnp.int32, (be, 128), 1)
    acc0 = jnp.zeros((be, 128), jnp.float32)
    acc1 = jnp.zeros((be, 128), jnp.float32)
    for g in range(nchunk):
        wg = (
            jnp.dot(hcat, w2_ref[:, g * 128:(g + 1) * 128],
                    preferred_element_type=jnp.float32)
            + b2_ref[:, g * 128:(g + 1) * 128]
        )

        def sel(xj):
            i0 = g * gi
            v = xj[:, i0 + gi - 1:i0 + gi]
            for t in range(gi - 2, -1, -1):
                v = jnp.where(lane < (t + 1) * cout, xj[:, i0 + t:i0 + t + 1], v)
            return v

        acc0 += sel(xj0) * wg
        acc1 += sel(xj1) * wg
    w = 128
    while w > cout:
        w //= 2
        acc0 = acc0[:, :w] + acc0[:, w:2 * w]
        acc1 = acc1[:, :w] + acc1[:, w:2 * w]
    out_ref[0] = acc0
    out_ref[1] = acc1


def _msg_call(ea, w1, b1, w2, b2, xj, cin, cout, be=256):
    nblk = _E // be
    body = functools.partial(_msg_body, cin=cin, cout=cout, be=be)
    return pl.pallas_call(
        body,
        grid=(nblk,),
        in_specs=[
            pl.BlockSpec((be, 5), lambda k: _i32map(k, 0)),
            pl.BlockSpec((5, 128), lambda k: _i32map(0, 0)),
            pl.BlockSpec((1, 128), lambda k: _i32map(0, 0)),
            pl.BlockSpec((384, cin * cout), lambda k: _i32map(0, 0)),
            pl.BlockSpec((1, cin * cout), lambda k: _i32map(0, 0)),
            pl.BlockSpec((1, be, cin), lambda k: _i32map(0, k, 0)),
            pl.BlockSpec((1, be, cin), lambda k: _i32map(1, k, 0)),
        ],
        out_specs=pl.BlockSpec((2, be, cout), lambda k: _i32map(0, k, 0)),
        out_shape=jax.ShapeDtypeStruct((2, _E, cout), jnp.float32),
    )(ea, w1, b1, w2, b2, xj, xj)


def _epi_call(part, hprev, root, bias, cin, cout, bn=2048):
    def body(p_ref, h_ref, r_ref, b_ref, o_ref):
        s = (
            p_ref[...]
            + jnp.dot(h_ref[...], r_ref[...], preferred_element_type=jnp.float32,
                    precision=lax.Precision.HIGHEST)
            + b_ref[...]
        )
        o_ref[...] = jnp.where(s > 0, s, jnp.exp(s) - 1.0)

    return pl.pallas_call(
        body,
        grid=(_T // bn,),
        in_specs=[
            pl.BlockSpec((bn, cout), lambda k: _i32map(k, 0)),
            pl.BlockSpec((bn, cin), lambda k: _i32map(k, 0)),
            pl.BlockSpec((cin, cout), lambda k: _i32map(0, 0)),
            pl.BlockSpec((1, cout), lambda k: _i32map(0, 0)),
        ],
        out_specs=pl.BlockSpec((bn, cout), lambda k: _i32map(k, 0)),
        out_shape=jax.ShapeDtypeStruct((_T, cout), jnp.float32),
    )(part, hprev, root, bias)


def _readout_call(h3, batch3, fc1_W, fc1_b, fc2_W, fc2_b, fc3_W, fc3_b, nb=1024):
    nblk = _N // nb

    def body(h0_ref, h1_ref, b_ref, w1_ref, b1_ref, w2_ref, b2_ref, w3_ref,
             b3_ref, o_ref, acc_ref, cnt_ref):
        k = pl.program_id(0)

        @pl.when(k == 0)
        def _():
            acc_ref[...] = jnp.zeros_like(acc_ref)
            cnt_ref[...] = jnp.zeros_like(cnt_ref)

        hm = (h0_ref[0] + h1_ref[0]) * 0.5
        brow = b_ref[0]
        gids = lax.broadcasted_iota(jnp.int32, (_NGRAPH, nb), 0)
        oh = (gids == brow).astype(jnp.float32)
        acc_ref[...] += jnp.dot(oh, hm, preferred_element_type=jnp.float32,
                    precision=lax.Precision.HIGHEST)
        cnt_ref[...] += jnp.dot(oh, jnp.ones((nb, 64), jnp.float32),
                                preferred_element_type=jnp.float32,
                    precision=lax.Precision.HIGHEST)

        @pl.when(k == nblk - 1)
        def _():
            g = acc_ref[...] / jnp.maximum(cnt_ref[...], 1.0)
            g1 = jnp.dot(g, w1_ref[...], preferred_element_type=jnp.float32,
                    precision=lax.Precision.HIGHEST) + b1_ref[...]
            g1 = jnp.where(g1 > 0, g1, jnp.exp(g1) - 1.0)
            g2 = jnp.dot(g1, w2_ref[...], preferred_element_type=jnp.float32,
                    precision=lax.Precision.HIGHEST) + b2_ref[...]
            g2 = jnp.where(g2 > 0, g2, jnp.exp(g2) - 1.0)
            o_ref[...] = jnp.dot(g2, w3_ref[...], preferred_element_type=jnp.float32,
                    precision=lax.Precision.HIGHEST) + b3_ref[...]

    return pl.pallas_call(
        body,
        grid=(nblk,),
        in_specs=[
            pl.BlockSpec((1, nb, 64), lambda k: _i32map(0, k, 0)),
            pl.BlockSpec((1, nb, 64), lambda k: _i32map(1, k, 0)),
            pl.BlockSpec((1, 1, nb), lambda k: _i32map(k, 0, 0)),
            pl.BlockSpec((64, 32), lambda k: _i32map(0, 0)),
            pl.BlockSpec((1, 32), lambda k: _i32map(0, 0)),
            pl.BlockSpec((32, 16), lambda k: _i32map(0, 0)),
            pl.BlockSpec((1, 16), lambda k: _i32map(0, 0)),
            pl.BlockSpec((16, 1), lambda k: _i32map(0, 0)),
            pl.BlockSpec((1, 1), lambda k: _i32map(0, 0)),
        ],
        out_specs=pl.BlockSpec((_NGRAPH, 1), lambda k: _i32map(0, 0)),
        out_shape=jax.ShapeDtypeStruct((_NGRAPH, 1), jnp.float32),
        scratch_shapes=[
            pltpu.VMEM((_NGRAPH, 64), jnp.float32),
            pltpu.VMEM((_NGRAPH, 64), jnp.float32),
        ],
    )(h3, h3, batch3, fc1_W, fc1_b, fc2_W, fc2_b, fc3_W, fc3_b)


def kernel(x, edge_index, edge_attr, batch, nn1_W1, nn1_b1, nn1_W2, nn1_b2,
           root1, bias1, nn2_W1, nn2_b1, nn2_W2, nn2_b2, root2, bias2,
           nn3_W1, nn3_b1, nn3_W2, nn3_b2, root3, bias3, fc1_W, fc1_b,
           fc2_W, fc2_b, fc3_W, fc3_b):
    f32 = jnp.float32
    ei = edge_index.astype(jnp.int32)
    off = (jnp.max(edge_index) + 1).astype(jnp.int32)
    src = jnp.concatenate([ei[0], ei[0] + off]).reshape(_NW, _J, _CHUNK)
    dst = jnp.concatenate([ei[1], ei[1] + off]).reshape(_NS, _J2, _CHUNK)

    drop = jax.random.bernoulli(jax.random.key(42), _P, (_RUNS, _N))
    xr = jnp.where(
        drop[..., None], 0.0, jnp.broadcast_to(x[None], (_RUNS, _N, _FEAT))
    ).reshape(_T, _FEAT).astype(f32)

    ea = edge_attr.astype(f32)
    zeros32 = jnp.zeros((_STRIPE2, 32), f32)
    zeros64 = jnp.zeros((_STRIPE2, 64), f32)

    def layer(h, W1, b1, W2, b2, root, bias, cin, cout, zeros):
        xjr = _gather_fn(cin)(h, src)
        xj = xjr.reshape(_RUNS, _E, cin)
        W2f = W2.astype(f32)
        w2h = W2f.astype(jnp.bfloat16)
        w2l = (W2f - w2h.astype(f32)).astype(jnp.bfloat16)
        w2cat = jnp.concatenate([w2h, w2h, w2l], axis=0)
        msg = _msg_call(ea, W1.astype(f32), b1.reshape(1, 128).astype(f32),
                        w2cat, b2.reshape(1, -1).astype(f32),
                        xj, cin, cout)
        part = _scatter_fn(cout)(msg.reshape(_NS * _J2, _CHUNK, cout), dst, zeros)
        return _epi_call(part, h, root.astype(f32),
                         bias.reshape(1, cout).astype(f32), cin, cout)

    h1 = layer(xr, nn1_W1, nn1_b1, nn1_W2, nn1_b2, root1, bias1, _FEAT, 32, zeros32)
    h2 = layer(h1, nn2_W1, nn2_b1, nn2_W2, nn2_b2, root2, bias2, 32, 64, zeros64)
    h3 = layer(h2, nn3_W1, nn3_b1, nn3_W2, nn3_b2, root3, bias3, 64, 64, zeros64)

    g = _readout_call(
        h3.reshape(_RUNS, _N, 64),
        batch.astype(jnp.int32).reshape(_N // 1024, 1, 1024),
        fc1_W.astype(f32), fc1_b.reshape(1, 32).astype(f32),
        fc2_W.astype(f32), fc2_b.reshape(1, 16).astype(f32),
        fc3_W.astype(f32), fc3_b.reshape(1, 1).astype(f32),
    )
    return g.reshape(-1).astype(nn1_W1.dtype)


# speedup vs baseline: 242.0680x; 1.0400x over previous
import functools

import jax
import jax.numpy as jnp
from jax import lax
from jax.experimental import pallas as pl
from jax.experimental.pallas import tpu as pltpu
from jax.experimental.pallas import tpu_sc as plsc

_N = 10240
_E = 20480
_FEAT = 32
_RUNS = 2
_P = 2.0 / (1.0 + _RUNS)
_NGRAPH = 512
_NC = 2
_NS = 16
_NW = _NC * _NS
_CHUNK = 128
_EDGES_ALL = _RUNS * _E
_EPW = _EDGES_ALL // _NW
_J = _EPW // _CHUNK
_T = _RUNS * _N
_HALF = _T // _NC
_J2 = _EDGES_ALL // _NS // _CHUNK
_STRIPE2 = _HALF // _NS
_BE = 256


def _i32map(*vals):
    return tuple(jnp.int32(v) for v in vals)


def _elu(x):
    return jnp.where(x > 0, x, jnp.exp(x) - 1.0)


@functools.cache
def _gather_fn(cin):
    mesh = plsc.VectorSubcoreMesh(core_axis_name="c", subcore_axis_name="s")

    @functools.partial(
        pl.kernel,
        out_type=jax.ShapeDtypeStruct((_NW, _J, _CHUNK, cin), jnp.float32),
        mesh=mesh,
        compiler_params=pltpu.CompilerParams(use_tc_tiling_on_sc=False),
        scratch_types=[
            pltpu.VMEM((_J, _CHUNK), jnp.int32),
            pltpu.VMEM((_J, _CHUNK, cin), jnp.float32),
            pltpu.SemaphoreType.DMA,
        ],
    )
    def gather(table_hbm, idx_hbm, out_hbm, idx_v, rows_v, sem):
        wid = lax.axis_index("s") * _NC + lax.axis_index("c")
        pltpu.sync_copy(idx_hbm.at[wid], idx_v)
        descs = [
            pltpu.async_copy(
                table_hbm.at[idx_v.at[jnp.int32(j)]], rows_v.at[jnp.int32(j)], sem)
            for j in range(_J)
        ]
        for d in descs:
            d.wait()
        pltpu.sync_copy(rows_v, out_hbm.at[wid])

    return gather


@functools.cache
def _scatter_fn(cout):
    mesh = plsc.VectorSubcoreMesh(core_axis_name="c", subcore_axis_name="s")

    @functools.partial(
        pl.kernel,
        out_type=jax.ShapeDtypeStruct((_T, cout), jnp.float32),
        mesh=mesh,
        compiler_params=pltpu.CompilerParams(use_tc_tiling_on_sc=False),
        scratch_types=[
            pltpu.VMEM((_J2, _CHUNK), jnp.int32),
            pltpu.VMEM((_CHUNK, cout), jnp.float32),
            pltpu.VMEM_SHARED((_HALF + 8, cout), jnp.float32),
            pltpu.SemaphoreType.DMA,
        ],
    )
    def scatter(msg_hbm, idx_hbm, rt_hbm, out_hbm, idx_v, msg_v, acc_sh, sem):
        c = lax.axis_index("c")
        s = lax.axis_index("s")
        base = c * _HALF
        pltpu.sync_copy(
            rt_hbm.at[pl.ds(base + s * _STRIPE2, _STRIPE2)],
            acc_sh.at[pl.ds(s * _STRIPE2, _STRIPE2)],
        )
        pltpu.sync_copy(idx_hbm.at[s], idx_v)
        for j in range(_J2):
            for i in range(_CHUNK // 16):
                v = idx_v[jnp.int32(j), pl.ds(i * 16, 16)] - base
                ok = (v >= 0) & (v < _HALF)
                idx_v[jnp.int32(j), pl.ds(i * 16, 16)] = jnp.where(
                    ok, v, jnp.int32(_HALF))
        plsc.subcore_barrier()
        for j in range(_J2):
            pltpu.sync_copy(msg_hbm.at[s * _J2 + j], msg_v)
            pltpu.async_copy(
                msg_v, acc_sh.at[idx_v.at[jnp.int32(j)]], sem, add=True
            ).wait()
        plsc.subcore_barrier()
        pltpu.sync_copy(
            acc_sh.at[pl.ds(s * _STRIPE2, _STRIPE2)],
            out_hbm.at[pl.ds(base + s * _STRIPE2, _STRIPE2)],
        )

    return scatter


def _msg_body(ea_ref, w1_ref, b1_ref, w2_ref, b2_ref, xj0_ref, xj1_ref,
              h_ref, root_ref, rb_ref, out_ref, rt_ref, *, cin, cout, act):
    gi = 128 // cout
    nchunk = cin // gi
    h1 = jnp.maximum(
        jnp.dot(ea_ref[...], w1_ref[...], preferred_element_type=jnp.float32,
                precision=lax.Precision.HIGHEST)
        + b1_ref[...],
        0.0,
    )
    xj0 = xj0_ref[0]
    xj1 = xj1_ref[0]
    hp = h_ref[...]
    if act:
        xj0 = _elu(xj0)
        xj1 = _elu(xj1)
        hp = _elu(hp)
    rt_ref[...] = (
        jnp.dot(hp, root_ref[...], preferred_element_type=jnp.float32,
                precision=lax.Precision.HIGHEST)
        + rb_ref[...]
    )
    lane = lax.broadcasted_iota(jnp.int32, (_BE, 128), 1)
    acc0 = jnp.zeros((_BE, 128), jnp.float32)
    acc1 = jnp.zeros((_BE, 128), jnp.float32)
    for g in range(nchunk):
        wg = (
            jnp.dot(h1, w2_ref[:, g * 128:(g + 1) * 128],
                    preferred_element_type=jnp.float32,
                    precision=lax.Precision.HIGHEST)
            + b2_ref[:, g * 128:(g + 1) * 128]
        )

        def sel(xj):
            i0 = g * gi
            v = xj[:, i0 + gi - 1:i0 + gi]
            for t in range(gi - 2, -1, -1):
                v = jnp.where(lane < (t + 1) * cout, xj[:, i0 + t:i0 + t + 1], v)
            return v

        acc0 += sel(xj0) * wg
        acc1 += sel(xj1) * wg
    w = 128
    while w > cout:
        w //= 2
        acc0 = acc0[:, :w] + acc0[:, w:2 * w]
        acc1 = acc1[:, :w] + acc1[:, w:2 * w]
    out_ref[0] = acc0
    out_ref[1] = acc1


def _msg_call(ea, w1, b1, w2, b2, xj, hprev, root, rbias, cin, cout, act):
    nblk = _E // _BE
    body = functools.partial(_msg_body, cin=cin, cout=cout, act=act)
    return pl.pallas_call(
        body,
        grid=(nblk,),
        in_specs=[
            pl.BlockSpec((_BE, 5), lambda k: _i32map(k, 0)),
            pl.BlockSpec((5, 128), lambda k: _i32map(0, 0)),
            pl.BlockSpec((1, 128), lambda k: _i32map(0, 0)),
            pl.BlockSpec((128, cin * cout), lambda k: _i32map(0, 0)),
            pl.BlockSpec((1, cin * cout), lambda k: _i32map(0, 0)),
            pl.BlockSpec((1, _BE, cin), lambda k: _i32map(0, k, 0)),
            pl.BlockSpec((1, _BE, cin), lambda k: _i32map(1, k, 0)),
            pl.BlockSpec((_BE, cin), lambda k: _i32map(k, 0)),
            pl.BlockSpec((cin, cout), lambda k: _i32map(0, 0)),
            pl.BlockSpec((1, cout), lambda k: _i32map(0, 0)),
        ],
        out_specs=[
            pl.BlockSpec((2, _BE, cout), lambda k: _i32map(0, k, 0)),
            pl.BlockSpec((_BE, cout), lambda k: _i32map(k, 0)),
        ],
        out_shape=[
            jax.ShapeDtypeStruct((2, _E, cout), jnp.float32),
            jax.ShapeDtypeStruct((_T, cout), jnp.float32),
        ],
    )(ea, w1, b1, w2, b2, xj, xj, hprev, root, rbias)


def _readout_call(h3, batch3, fc1_W, fc1_b, fc2_W, fc2_b, fc3_W, fc3_b, nb=1024):
    nblk = _N // nb

    def body(h0_ref, h1_ref, b_ref, w1_ref, b1_ref, w2_ref, b2_ref, w3_ref,
             b3_ref, o_ref, acc_ref, cnt_ref):
        k = pl.program_id(0)

        @pl.when(k == 0)
        def _():
            acc_ref[...] = jnp.zeros_like(acc_ref)
            cnt_ref[...] = jnp.zeros_like(cnt_ref)

        hm = (_elu(h0_ref[0]) + _elu(h1_ref[0])) * 0.5
        brow = b_ref[0]
        gids = lax.broadcasted_iota(jnp.int32, (_NGRAPH, nb), 0)
        oh = (gids == brow).astype(jnp.float32)
        acc_ref[...] += jnp.dot(oh, hm, preferred_element_type=jnp.float32,
                                precision=lax.Precision.HIGHEST)
        cnt_ref[...] += jnp.dot(oh, jnp.ones((nb, 64), jnp.float32),
                                preferred_element_type=jnp.float32)

        @pl.when(k == nblk - 1)
        def _():
            g = acc_ref[...] / jnp.maximum(cnt_ref[...], 1.0)
            g1 = jnp.dot(g, w1_ref[...], preferred_element_type=jnp.float32,
                         precision=lax.Precision.HIGHEST) + b1_ref[...]
            g1 = _elu(g1)
            g2 = jnp.dot(g1, w2_ref[...], preferred_element_type=jnp.float32,
                         precision=lax.Precision.HIGHEST) + b2_ref[...]
            g2 = _elu(g2)
            o_ref[...] = jnp.dot(
                g2, w3_ref[...], preferred_element_type=jnp.float32,
                precision=lax.Precision.HIGHEST) + b3_ref[...]

    return pl.pallas_call(
        body,
        grid=(nblk,),
        in_specs=[
            pl.BlockSpec((1, nb, 64), lambda k: _i32map(0, k, 0)),
            pl.BlockSpec((1, nb, 64), lambda k: _i32map(1, k, 0)),
            pl.BlockSpec((1, 1, nb), lambda k: _i32map(k, 0, 0)),
            pl.BlockSpec((64, 32), lambda k: _i32map(0, 0)),
            pl.BlockSpec((1, 32), lambda k: _i32map(0, 0)),
            pl.BlockSpec((32, 16), lambda k: _i32map(0, 0)),
            pl.BlockSpec((1, 16), lambda k: _i32map(0, 0)),
            pl.BlockSpec((16, 1), lambda k: _i32map(0, 0)),
            pl.BlockSpec((1, 1), lambda k: _i32map(0, 0)),
        ],
        out_specs=pl.BlockSpec((_NGRAPH, 1), lambda k: _i32map(0, 0)),
        out_shape=jax.ShapeDtypeStruct((_NGRAPH, 1), jnp.float32),
        scratch_shapes=[
            pltpu.VMEM((_NGRAPH, 64), jnp.float32),
            pltpu.VMEM((_NGRAPH, 64), jnp.float32),
        ],
    )(h3, h3, batch3, fc1_W, fc1_b, fc2_W, fc2_b, fc3_W, fc3_b)


def kernel(x, edge_index, edge_attr, batch, nn1_W1, nn1_b1, nn1_W2, nn1_b2,
           root1, bias1, nn2_W1, nn2_b1, nn2_W2, nn2_b2, root2, bias2,
           nn3_W1, nn3_b1, nn3_W2, nn3_b2, root3, bias3, fc1_W, fc1_b,
           fc2_W, fc2_b, fc3_W, fc3_b):
    f32 = jnp.float32
    ei = edge_index.astype(jnp.int32)
    off = (jnp.max(edge_index) + 1).astype(jnp.int32)
    src = jnp.concatenate([ei[0], ei[0] + off]).reshape(_NW, _J, _CHUNK)
    dst = jnp.concatenate([ei[1], ei[1] + off]).reshape(_NS, _J2, _CHUNK)

    drop = jax.random.bernoulli(jax.random.key(42), _P, (_RUNS, _N))
    xr = jnp.where(
        drop[..., None], 0.0, jnp.broadcast_to(x[None], (_RUNS, _N, _FEAT))
    ).reshape(_T, _FEAT).astype(f32)

    ea = edge_attr.astype(f32)

    def layer(h, W1, b1, W2, b2, root, bias, cin, cout, act):
        xjr = _gather_fn(cin)(h, src)
        xj = xjr.reshape(_RUNS, _E, cin)
        msg, rt = _msg_call(ea, W1.astype(f32), b1.reshape(1, 128).astype(f32),
                            W2.astype(f32), b2.reshape(1, -1).astype(f32),
                            xj, h, root.astype(f32),
                            bias.reshape(1, cout).astype(f32), cin, cout, act)
        return _scatter_fn(cout)(msg.reshape(_NS * _J2, _CHUNK, cout), dst, rt)

    s1 = layer(xr, nn1_W1, nn1_b1, nn1_W2, nn1_b2, root1, bias1, _FEAT, 32, False)
    s2 = layer(s1, nn2_W1, nn2_b1, nn2_W2, nn2_b2, root2, bias2, 32, 64, True)
    s3 = layer(s2, nn3_W1, nn3_b1, nn3_W2, nn3_b2, root3, bias3, 64, 64, True)

    g = _readout_call(
        s3.reshape(_RUNS, _N, 64),
        batch.astype(jnp.int32).reshape(_N // 1024, 1, 1024),
        fc1_W.astype(f32), fc1_b.reshape(1, 32).astype(f32),
        fc2_W.astype(f32), fc2_b.reshape(1, 16).astype(f32),
        fc3_W.astype(f32), fc3_b.reshape(1, 1).astype(f32),
    )
    return g.reshape(-1).astype(nn1_W1.dtype)


# speedup vs baseline: 243.5903x; 1.0063x over previous
import functools

import jax
import jax.numpy as jnp
from jax import lax
from jax.experimental import pallas as pl
from jax.experimental.pallas import tpu as pltpu
from jax.experimental.pallas import tpu_sc as plsc

_N = 10240
_E = 20480
_FEAT = 32
_RUNS = 2
_P = 2.0 / (1.0 + _RUNS)
_NGRAPH = 512
_NC = 2
_NS = 16
_NW = _NC * _NS
_CHUNK = 128
_EDGES_ALL = _RUNS * _E
_EPW = _EDGES_ALL // _NW
_J = _EPW // _CHUNK
_T = _RUNS * _N
_HALF = _T // _NC
_J2 = _EDGES_ALL // _NS // _CHUNK
_STRIPE2 = _HALF // _NS
_BE = 256


def _i32map(*vals):
    return tuple(jnp.int32(v) for v in vals)


def _elu(x):
    return jnp.where(x > 0, x, jnp.exp(x) - 1.0)


@functools.cache
def _gather_fn(cin):
    mesh = plsc.VectorSubcoreMesh(core_axis_name="c", subcore_axis_name="s")

    @functools.partial(
        pl.kernel,
        out_type=jax.ShapeDtypeStruct((_NW, _J, _CHUNK, cin), jnp.float32),
        mesh=mesh,
        compiler_params=pltpu.CompilerParams(use_tc_tiling_on_sc=False),
        scratch_types=[
            pltpu.VMEM((_J, _CHUNK), jnp.int32),
            pltpu.VMEM((_J, _CHUNK, cin), jnp.float32),
            pltpu.SemaphoreType.DMA,
        ],
    )
    def gather(table_hbm, idx_hbm, out_hbm, idx_v, rows_v, sem):
        wid = lax.axis_index("s") * _NC + lax.axis_index("c")
        pltpu.sync_copy(idx_hbm.at[wid], idx_v)
        descs = [
            pltpu.async_copy(
                table_hbm.at[idx_v.at[jnp.int32(j)]], rows_v.at[jnp.int32(j)], sem)
            for j in range(_J)
        ]
        for d in descs:
            d.wait()
        pltpu.sync_copy(rows_v, out_hbm.at[wid])

    return gather


@functools.cache
def _scatter_fn(cout):
    mesh = plsc.VectorSubcoreMesh(core_axis_name="c", subcore_axis_name="s")

    @functools.partial(
        pl.kernel,
        out_type=jax.ShapeDtypeStruct((_T, cout), jnp.float32),
        mesh=mesh,
        compiler_params=pltpu.CompilerParams(use_tc_tiling_on_sc=False),
        scratch_types=[
            pltpu.VMEM((_J2, _CHUNK), jnp.int32),
            pltpu.VMEM((2, _CHUNK, cout), jnp.float32),
            pltpu.VMEM_SHARED((_HALF + 8, cout), jnp.float32),
            pltpu.SemaphoreType.DMA,
            pltpu.SemaphoreType.DMA,
        ],
    )
    def scatter(msg_hbm, idx_hbm, rt_hbm, out_hbm, idx_v, msg_v, acc_sh,
                sem, sem2):
        c = lax.axis_index("c")
        s = lax.axis_index("s")
        base = c * _HALF
        rt_d = pltpu.async_copy(
            rt_hbm.at[pl.ds(base + s * _STRIPE2, _STRIPE2)],
            acc_sh.at[pl.ds(s * _STRIPE2, _STRIPE2)],
            sem,
        )
        ld = pltpu.async_copy(
            msg_hbm.at[s * _J2], msg_v.at[jnp.int32(0)], sem2)
        pltpu.sync_copy(idx_hbm.at[s], idx_v)
        for j in range(_J2):
            for i in range(_CHUNK // 16):
                v = idx_v[jnp.int32(j), pl.ds(i * 16, 16)] - base
                ok = (v >= 0) & (v < _HALF)
                idx_v[jnp.int32(j), pl.ds(i * 16, 16)] = jnp.where(
                    ok, v, jnp.int32(_HALF))
        rt_d.wait()
        plsc.subcore_barrier()
        for j in range(_J2):
            ld.wait()
            d = pltpu.async_copy(
                msg_v.at[jnp.int32(j % 2)],
                acc_sh.at[idx_v.at[jnp.int32(j)]], sem, add=True)
            if j + 1 < _J2:
                ld = pltpu.async_copy(
                    msg_hbm.at[s * _J2 + j + 1],
                    msg_v.at[jnp.int32((j + 1) % 2)], sem2)
            d.wait()
        plsc.subcore_barrier()
        pltpu.sync_copy(
            acc_sh.at[pl.ds(s * _STRIPE2, _STRIPE2)],
            out_hbm.at[pl.ds(base + s * _STRIPE2, _STRIPE2)],
        )

    return scatter


def _msg_body(ea_ref, w1_ref, b1_ref, w2_ref, b2_ref, xj0_ref, xj1_ref,
              h_ref, root_ref, rb_ref, out_ref, rt_ref, *, cin, cout, act):
    gi = 128 // cout
    nchunk = cin // gi
    h1 = jnp.maximum(
        jnp.dot(ea_ref[...], w1_ref[...], preferred_element_type=jnp.float32,
                precision=lax.Precision.HIGHEST)
        + b1_ref[...],
        0.0,
    )
    xj0 = xj0_ref[0]
    xj1 = xj1_ref[0]
    hp = h_ref[...]
    if act:
        xj0 = _elu(xj0)
        xj1 = _elu(xj1)
        hp = _elu(hp)
    rt_ref[...] = (
        jnp.dot(hp, root_ref[...], preferred_element_type=jnp.float32,
                precision=lax.Precision.HIGHEST)
        + rb_ref[...]
    )
    lane = lax.broadcasted_iota(jnp.int32, (_BE, 128), 1)
    acc0 = jnp.zeros((_BE, 128), jnp.float32)
    acc1 = jnp.zeros((_BE, 128), jnp.float32)
    for g in range(nchunk):
        wg = (
            jnp.dot(h1, w2_ref[:, g * 128:(g + 1) * 128],
                    preferred_element_type=jnp.float32,
                    precision=lax.Precision.HIGHEST)
            + b2_ref[:, g * 128:(g + 1) * 128]
        )

        def sel(xj):
            i0 = g * gi
            v = xj[:, i0 + gi - 1:i0 + gi]
            for t in range(gi - 2, -1, -1):
                v = jnp.where(lane < (t + 1) * cout, xj[:, i0 + t:i0 + t + 1], v)
            return v

        acc0 += sel(xj0) * wg
        acc1 += sel(xj1) * wg
    w = 128
    while w > cout:
        w //= 2
        acc0 = acc0[:, :w] + acc0[:, w:2 * w]
        acc1 = acc1[:, :w] + acc1[:, w:2 * w]
    out_ref[0] = acc0
    out_ref[1] = acc1


def _msg_call(ea, w1, b1, w2, b2, xj, hprev, root, rbias, cin, cout, act):
    nblk = _E // _BE
    body = functools.partial(_msg_body, cin=cin, cout=cout, act=act)
    return pl.pallas_call(
        body,
        grid=(nblk,),
        in_specs=[
            pl.BlockSpec((_BE, 5), lambda k: _i32map(k, 0)),
            pl.BlockSpec((5, 128), lambda k: _i32map(0, 0)),
            pl.BlockSpec((1, 128), lambda k: _i32map(0, 0)),
            pl.BlockSpec((128, cin * cout), lambda k: _i32map(0, 0)),
            pl.BlockSpec((1, cin * cout), lambda k: _i32map(0, 0)),
            pl.BlockSpec((1, _BE, cin), lambda k: _i32map(0, k, 0)),
            pl.BlockSpec((1, _BE, cin), lambda k: _i32map(1, k, 0)),
            pl.BlockSpec((_BE, cin), lambda k: _i32map(k, 0)),
            pl.BlockSpec((cin, cout), lambda k: _i32map(0, 0)),
            pl.BlockSpec((1, cout), lambda k: _i32map(0, 0)),
        ],
        out_specs=[
            pl.BlockSpec((2, _BE, cout), lambda k: _i32map(0, k, 0)),
            pl.BlockSpec((_BE, cout), lambda k: _i32map(k, 0)),
        ],
        out_shape=[
            jax.ShapeDtypeStruct((2, _E, cout), jnp.float32),
            jax.ShapeDtypeStruct((_T, cout), jnp.float32),
        ],
    )(ea, w1, b1, w2, b2, xj, xj, hprev, root, rbias)


def _readout_call(h3, batch3, fc1_W, fc1_b, fc2_W, fc2_b, fc3_W, fc3_b, nb=1024):
    nblk = _N // nb

    def body(h0_ref, h1_ref, b_ref, w1_ref, b1_ref, w2_ref, b2_ref, w3_ref,
             b3_ref, o_ref, acc_ref, cnt_ref):
        k = pl.program_id(0)

        @pl.when(k == 0)
        def _():
            acc_ref[...] = jnp.zeros_like(acc_ref)
            cnt_ref[...] = jnp.zeros_like(cnt_ref)

        hm = (_elu(h0_ref[0]) + _elu(h1_ref[0])) * 0.5
        brow = b_ref[0]
        gids = lax.broadcasted_iota(jnp.int32, (_NGRAPH, nb), 0)
        oh = (gids == brow).astype(jnp.float32)
        acc_ref[...] += jnp.dot(oh, hm, preferred_element_type=jnp.float32,
                                precision=lax.Precision.HIGHEST)
        cnt_ref[...] += jnp.dot(oh, jnp.ones((nb, 64), jnp.float32),
                                preferred_element_type=jnp.float32)

        @pl.when(k == nblk - 1)
        def _():
            g = acc_ref[...] / jnp.maximum(cnt_ref[...], 1.0)
            g1 = jnp.dot(g, w1_ref[...], preferred_element_type=jnp.float32,
                         precision=lax.Precision.HIGHEST) + b1_ref[...]
            g1 = _elu(g1)
            g2 = jnp.dot(g1, w2_ref[...], preferred_element_type=jnp.float32,
                         precision=lax.Precision.HIGHEST) + b2_ref[...]
            g2 = _elu(g2)
            o_ref[...] = jnp.dot(
                g2, w3_ref[...], preferred_element_type=jnp.float32,
                precision=lax.Precision.HIGHEST) + b3_ref[...]

    return pl.pallas_call(
        body,
        grid=(nblk,),
        in_specs=[
            pl.BlockSpec((1, nb, 64), lambda k: _i32map(0, k, 0)),
            pl.BlockSpec((1, nb, 64), lambda k: _i32map(1, k, 0)),
            pl.BlockSpec((1, 1, nb), lambda k: _i32map(k, 0, 0)),
            pl.BlockSpec((64, 32), lambda k: _i32map(0, 0)),
            pl.BlockSpec((1, 32), lambda k: _i32map(0, 0)),
            pl.BlockSpec((32, 16), lambda k: _i32map(0, 0)),
            pl.BlockSpec((1, 16), lambda k: _i32map(0, 0)),
            pl.BlockSpec((16, 1), lambda k: _i32map(0, 0)),
            pl.BlockSpec((1, 1), lambda k: _i32map(0, 0)),
        ],
        out_specs=pl.BlockSpec((_NGRAPH, 1), lambda k: _i32map(0, 0)),
        out_shape=jax.ShapeDtypeStruct((_NGRAPH, 1), jnp.float32),
        scratch_shapes=[
            pltpu.VMEM((_NGRAPH, 64), jnp.float32),
            pltpu.VMEM((_NGRAPH, 64), jnp.float32),
        ],
    )(h3, h3, batch3, fc1_W, fc1_b, fc2_W, fc2_b, fc3_W, fc3_b)


def kernel(x, edge_index, edge_attr, batch, nn1_W1, nn1_b1, nn1_W2, nn1_b2,
           root1, bias1, nn2_W1, nn2_b1, nn2_W2, nn2_b2, root2, bias2,
           nn3_W1, nn3_b1, nn3_W2, nn3_b2, root3, bias3, fc1_W, fc1_b,
           fc2_W, fc2_b, fc3_W, fc3_b):
    f32 = jnp.float32
    ei = edge_index.astype(jnp.int32)
    off = (jnp.max(edge_index) + 1).astype(jnp.int32)
    src = jnp.concatenate([ei[0], ei[0] + off]).reshape(_NW, _J, _CHUNK)
    dst = jnp.concatenate([ei[1], ei[1] + off]).reshape(_NS, _J2, _CHUNK)

    drop = jax.random.bernoulli(jax.random.key(42), _P, (_RUNS, _N))
    xr = jnp.where(
        drop[..., None], 0.0, jnp.broadcast_to(x[None], (_RUNS, _N, _FEAT))
    ).reshape(_T, _FEAT).astype(f32)

    ea = edge_attr.astype(f32)

    def layer(h, W1, b1, W2, b2, root, bias, cin, cout, act):
        xjr = _gather_fn(cin)(h, src)
        xj = xjr.reshape(_RUNS, _E, cin)
        msg, rt = _msg_call(ea, W1.astype(f32), b1.reshape(1, 128).astype(f32),
                            W2.astype(f32), b2.reshape(1, -1).astype(f32),
                            xj, h, root.astype(f32),
                            bias.reshape(1, cout).astype(f32), cin, cout, act)
        return _scatter_fn(cout)(msg.reshape(_NS * _J2, _CHUNK, cout), dst, rt)

    s1 = layer(xr, nn1_W1, nn1_b1, nn1_W2, nn1_b2, root1, bias1, _FEAT, 32, False)
    s2 = layer(s1, nn2_W1, nn2_b1, nn2_W2, nn2_b2, root2, bias2, 32, 64, True)
    s3 = layer(s2, nn3_W1, nn3_b1, nn3_W2, nn3_b2, root3, bias3, 64, 64, True)

    g = _readout_call(
        s3.reshape(_RUNS, _N, 64),
        batch.astype(jnp.int32).reshape(_N // 1024, 1, 1024),
        fc1_W.astype(f32), fc1_b.reshape(1, 32).astype(f32),
        fc2_W.astype(f32), fc2_b.reshape(1, 16).astype(f32),
        fc3_W.astype(f32), fc3_b.reshape(1, 1).astype(f32),
    )
    return g.reshape(-1).astype(nn1_W1.dtype)
